# Initial kernel scaffold; baseline (speedup 1.0000x reference)
import jax
import jax.numpy as jnp
from jax.experimental import pallas as pl


def kernel(x_hist, enc_misc, dec, edge_index, cheb_W, cheb_b, W_ih, W_hh, b_ih, b_hh, fc_W, fc_b):
    raise NotImplementedError("write your pallas kernel here")


# speedup vs baseline: 13.0470x; 13.0470x over previous
import functools

import jax
import jax.numpy as jnp
from jax.experimental import pallas as pl
from jax.experimental.pallas import tpu as pltpu


def _norm_kernel(d_ref, a_ref):
    d = d_ref[...]
    deg = jnp.sum(d, axis=0, keepdims=True)
    dis = jnp.where(deg > 0, 1.0 / jnp.sqrt(jnp.maximum(deg, 1e-12)), 0.0)
    a_ref[...] = -(dis * d) * jnp.transpose(dis)


def _prop_kernel(a_ref, x_ref, p_ref):
    p_ref[0] = jnp.dot(a_ref[...], x_ref[0],
                       preferred_element_type=jnp.float32, precision=jax.lax.Precision.HIGHEST)


def _rec_kernel(a_ref, xk_ref, pk_ref, fp_ref, pf_ref,
                w0_ref, w1_ref, wx_ref, wg_ref, whh_ref, fct_ref,
                bih_ref, bhh_ref, cb_ref, fb_ref,
                out_ref, h_ref, pr_ref, xf_ref, pw_ref, *, NK, T, HID):
    t = pl.program_id(1)
    f32 = jnp.float32

    @pl.when(t == 0)
    def _():
        h_ref[...] = jnp.zeros_like(h_ref)

    h = h_ref[...]

    @pl.when(t < NK)
    def _():
        xf_ref[...] = xk_ref[0, 0]
        pw_ref[...] = jnp.dot(pk_ref[0, 0], w1_ref[...],
                              preferred_element_type=f32, precision=jax.lax.Precision.HIGHEST)

    @pl.when(t >= NK)
    def _():
        xc8 = jnp.dot(h, fct_ref[...], preferred_element_type=f32, precision=jax.lax.Precision.HIGHEST) + fb_ref[...]
        xcur = xc8[:, 0:1]
        lane = jax.lax.broadcasted_iota(jnp.int32, xf_ref.shape, 1)
        xf_ref[...] = jnp.where(lane == 0, xcur, fp_ref[0, 0])
        pc8 = jnp.dot(a_ref[...], xc8, preferred_element_type=f32, precision=jax.lax.Precision.HIGHEST)
        pw_ref[...] = (jnp.dot(pf_ref[0, 0], w1_ref[...],
                               preferred_element_type=f32, precision=jax.lax.Precision.HIGHEST)
                       + pc8[:, 0:1] * w1_ref[0:1, :])

    xf = xf_ref[...]
    xg = jax.nn.sigmoid(jnp.dot(xf, w0_ref[...], preferred_element_type=f32, precision=jax.lax.Precision.HIGHEST)
                        + pw_ref[...] + cb_ref[...])
    gi = (jnp.dot(xf, wx_ref[...], preferred_element_type=f32, precision=jax.lax.Precision.HIGHEST)
          + jnp.dot(xg, wg_ref[...], preferred_element_type=f32, precision=jax.lax.Precision.HIGHEST)
          + bih_ref[...])
    gh = jnp.dot(h, whh_ref[...], preferred_element_type=f32, precision=jax.lax.Precision.HIGHEST) + bhh_ref[...]
    H = HID
    r = jax.nn.sigmoid(gi[:, :H] + gh[:, :H])
    z = jax.nn.sigmoid(gi[:, H:2 * H] + gh[:, H:2 * H])
    n = jnp.tanh(gi[:, 2 * H:] + r * gh[:, 2 * H:])
    hn = (1.0 - z) * n + z * h
    h_ref[...] = hn

    @pl.when(t >= NK - 1)
    def _():
        xo = jnp.dot(hn, fct_ref[...], preferred_element_type=f32, precision=jax.lax.Precision.HIGHEST) + fb_ref[...]
        lane8 = jax.lax.broadcasted_iota(jnp.int32, pr_ref.shape, 1)
        pr_ref[...] = jnp.where(lane8 == t - (NK - 1), xo[:, 0:1], pr_ref[...])

    @pl.when(t == T - 1)
    def _():
        out_ref[0] = pr_ref[...]


def kernel(x_hist, enc_misc, dec, edge_index, cheb_W, cheb_b,
           W_ih, W_hh, b_ih, b_hh, fc_W, fc_b):
    f32 = jnp.float32
    B, HIST, S, OUT = x_hist.shape
    FM = enc_misc.shape[-1]
    PRED = dec.shape[1]
    IN = OUT + FM
    GNN = cheb_W.shape[2]
    HID = W_hh.shape[1]
    NK = HIST
    NP = PRED - 1
    T = HIST - 1 + PRED
    SP = (S + 127) // 128 * 128
    CW = 16

    features = jnp.concatenate([enc_misc, dec], axis=1)

    xk = jnp.concatenate([x_hist, features[:, 1:NK + 1]], axis=-1)
    xk = jnp.pad(xk, ((0, 0), (0, 0), (0, SP - S), (0, CW - IN)))
    fp = jnp.pad(features[:, NK + 1:],
                 ((0, 0), (0, 0), (0, SP - S), (1, CW - FM - 1)))

    src, dst = edge_index[0], edge_index[1]
    w = (src != dst).astype(f32)
    D = jnp.zeros((SP, SP), f32).at[dst, src].add(w)

    A = pl.pallas_call(
        _norm_kernel,
        out_shape=jax.ShapeDtypeStruct((SP, SP), f32),
    )(D)

    C = (NK + NP) * CW
    Xbig = jnp.concatenate([
        xk.transpose(0, 2, 1, 3).reshape(B, SP, NK * CW),
        fp.transpose(0, 2, 1, 3).reshape(B, SP, NP * CW),
    ], axis=-1)
    Pbig = pl.pallas_call(
        _prop_kernel,
        grid=(B,),
        in_specs=[
            pl.BlockSpec((SP, SP), lambda b: (0, 0)),
            pl.BlockSpec((1, SP, C), lambda b: (b, 0, 0)),
        ],
        out_specs=pl.BlockSpec((1, SP, C), lambda b: (b, 0, 0)),
        out_shape=jax.ShapeDtypeStruct((B, SP, C), f32),
    )(A, Xbig)
    pk = Pbig[:, :, :NK * CW].reshape(B, SP, NK, CW).transpose(0, 2, 1, 3)
    pf = Pbig[:, :, NK * CW:].reshape(B, SP, NP, CW).transpose(0, 2, 1, 3)

    W0 = jnp.pad(cheb_W[0], ((0, CW - IN), (0, 0)))
    W1 = jnp.pad(cheb_W[1], ((0, CW - IN), (0, 0)))
    Wx = jnp.pad(W_ih[:, :IN].T, ((0, CW - IN), (0, 0)))
    Wg = W_ih[:, IN:].T
    Whh = W_hh.T
    fcT = jnp.pad(fc_W.T, ((0, 0), (0, 8 - OUT)))
    fb = jnp.pad(fc_b[None, :], ((0, 0), (0, 8 - OUT)))
    bih = b_ih[None, :]
    bhh = b_hh[None, :]
    cb = cheb_b[None, :]

    def full(shape):
        return pl.BlockSpec(shape, lambda b, t: (0,) * len(shape))

    def by_step(nmax, shift):
        def idx(b, t):
            return (b, jnp.clip(t - shift, 0, nmax - 1), 0, 0)
        return pl.BlockSpec((1, 1, SP, CW), idx)

    rec = functools.partial(_rec_kernel, NK=NK, T=T, HID=HID)
    out = pl.pallas_call(
        rec,
        grid=(B, T),
        in_specs=[
            full((SP, SP)),
            by_step(NK, 0),
            by_step(NK, 0),
            by_step(NP, NK),
            by_step(NP, NK),
            full(W0.shape), full(W1.shape), full(Wx.shape), full(Wg.shape),
            full(Whh.shape), full(fcT.shape),
            full(bih.shape), full(bhh.shape), full(cb.shape), full(fb.shape),
        ],
        out_specs=pl.BlockSpec((1, SP, 8), lambda b, t: (b, 0, 0)),
        out_shape=jax.ShapeDtypeStruct((B, SP, 8), f32),
        scratch_shapes=[
            pltpu.VMEM((SP, HID), f32),
            pltpu.VMEM((SP, 8), f32),
            pltpu.VMEM((SP, CW), f32),
            pltpu.VMEM((SP, GNN), f32),
        ],
    )(A, xk, pk, fp, pf, W0, W1, Wx, Wg, Whh, fcT, bih, bhh, cb, fb)

    preds = out[:, :S, :PRED]
    return preds.transpose(0, 2, 1)[..., None]


# speedup vs baseline: 33.9540x; 2.6024x over previous
import functools

import jax
import jax.numpy as jnp
from jax.experimental import pallas as pl
from jax.experimental.pallas import tpu as pltpu

_HI = jax.lax.Precision.HIGHEST


def _norm_kernel(dt_ref, at_ref):
    dt = dt_ref[...]
    deg = jnp.sum(dt, axis=1, keepdims=True)
    dis = jnp.where(deg > 0, 1.0 / jnp.sqrt(jnp.maximum(deg, 1e-12)), 0.0)
    at_ref[...] = -(dis * dt) * jnp.transpose(dis)


def _prop_kernel(at_ref, x_ref, p_ref):
    p_ref[0] = jnp.dot(x_ref[0], at_ref[...],
                       preferred_element_type=jnp.float32, precision=_HI)


def _rec_kernel(at_ref, x_ref, p_ref,
                w0_ref, w1_ref, wx_ref, wg_ref, whh_ref, fc_ref,
                bih_ref, bhh_ref, cb_ref, fb_ref,
                out_ref, h_ref, pr_ref, xct_ref, pct_ref, *, B, NK, T, HID):
    t = pl.program_id(0)
    f32 = jnp.float32
    SP = at_ref.shape[0]
    CW = x_ref.shape[2]
    pred = t >= NK

    @pl.when(t == 0)
    def _():
        h_ref[...] = jnp.zeros_like(h_ref)
        xct_ref[...] = jnp.zeros_like(xct_ref)
        pct_ref[...] = jnp.zeros_like(pct_ref)

    @pl.when(pred)
    def _():
        def fill(b, _):
            xc = (jnp.dot(fc_ref[...], h_ref[b], preferred_element_type=f32,
                          precision=_HI) + fb_ref[...])[0:1, :]
            xct_ref[pl.ds(b, 1), :] = xc
            return 0
        jax.lax.fori_loop(0, B, fill, 0)
        pct_ref[...] = jnp.dot(xct_ref[...], at_ref[...],
                               preferred_element_type=f32, precision=_HI)

    sub_x = jax.lax.broadcasted_iota(jnp.int32, (CW, SP), 0)
    sub8 = jax.lax.broadcasted_iota(jnp.int32, (8, SP), 0)
    i_out = t - (NK - 1)
    pm = jnp.where(pred, 1.0, 0.0).astype(f32)
    H = HID

    def body(b, _):
        h = h_ref[b]
        xb = x_ref[0, b]
        pb = p_ref[0, b]
        xcur = xct_ref[pl.ds(b, 1), :]
        pcb = pct_ref[pl.ds(b, 1), :]
        xf = jnp.where(jnp.logical_and(sub_x == 0, pred), xcur, xb)
        pw = (jnp.dot(w1_ref[...], pb, preferred_element_type=f32,
                      precision=_HI)
              + w1_ref[:, 0:1] * (pm * pcb))
        xg = jax.nn.sigmoid(
            jnp.dot(w0_ref[...], xf, preferred_element_type=f32, precision=_HI)
            + pw + cb_ref[...])
        gi = (jnp.dot(wx_ref[...], xf, preferred_element_type=f32,
                      precision=_HI)
              + jnp.dot(wg_ref[...], xg, preferred_element_type=f32,
                        precision=_HI)
              + bih_ref[...])
        gh = (jnp.dot(whh_ref[...], h, preferred_element_type=f32,
                      precision=_HI)
              + bhh_ref[...])
        r = jax.nn.sigmoid(gi[:H] + gh[:H])
        z = jax.nn.sigmoid(gi[H:2 * H] + gh[H:2 * H])
        n = jnp.tanh(gi[2 * H:] + r * gh[2 * H:])
        hn = (1.0 - z) * n + z * h
        h_ref[b] = hn
        xo = (jnp.dot(fc_ref[...], hn, preferred_element_type=f32,
                      precision=_HI) + fb_ref[...])[0:1, :]
        pr_ref[b] = jnp.where(sub8 == i_out, xo, pr_ref[b])
        return 0

    jax.lax.fori_loop(0, B, body, 0)

    @pl.when(t == T - 1)
    def _():
        out_ref[...] = pr_ref[...]


def kernel(x_hist, enc_misc, dec, edge_index, cheb_W, cheb_b,
           W_ih, W_hh, b_ih, b_hh, fc_W, fc_b):
    f32 = jnp.float32
    B, HIST, S, OUT = x_hist.shape
    FM = enc_misc.shape[-1]
    PRED = dec.shape[1]
    IN = OUT + FM
    GNN = cheb_W.shape[2]
    HID = W_hh.shape[1]
    NK = HIST
    NP = PRED - 1
    T = HIST - 1 + PRED
    SP = (S + 127) // 128 * 128
    CW = 16

    features = jnp.concatenate([enc_misc, dec], axis=1)

    xk = jnp.concatenate([x_hist, features[:, 1:NK + 1]], axis=-1)
    xk = jnp.pad(xk.transpose(1, 0, 3, 2),
                 ((0, 0), (0, 0), (0, CW - IN), (0, SP - S)))
    fp = jnp.pad(features[:, NK + 1:].transpose(1, 0, 3, 2),
                 ((0, 0), (0, 0), (1, CW - FM - 1), (0, SP - S)))
    xall = jnp.concatenate([xk, fp], axis=0)

    src, dst = edge_index[0], edge_index[1]
    w = (src != dst).astype(f32)
    DT = jnp.zeros((SP, SP), f32).at[src, dst].add(w)

    AT = pl.pallas_call(
        _norm_kernel,
        out_shape=jax.ShapeDtypeStruct((SP, SP), f32),
    )(DT)

    C = T * CW
    Xbig = xall.transpose(1, 0, 2, 3).reshape(B, C, SP)
    Pbig = pl.pallas_call(
        _prop_kernel,
        grid=(B,),
        in_specs=[
            pl.BlockSpec((SP, SP), lambda b: (0, 0)),
            pl.BlockSpec((1, C, SP), lambda b: (b, 0, 0)),
        ],
        out_specs=pl.BlockSpec((1, C, SP), lambda b: (b, 0, 0)),
        out_shape=jax.ShapeDtypeStruct((B, C, SP), f32),
    )(AT, Xbig)
    pall = Pbig.reshape(B, T, CW, SP).transpose(1, 0, 2, 3)

    W0 = jnp.pad(cheb_W[0].T, ((0, 0), (0, CW - IN)))
    W1 = jnp.pad(cheb_W[1].T, ((0, 0), (0, CW - IN)))
    Wx = jnp.pad(W_ih[:, :IN], ((0, 0), (0, CW - IN)))
    Wg = W_ih[:, IN:]
    Whh = W_hh
    fc = jnp.pad(fc_W, ((0, 8 - OUT), (0, 0)))
    fb = jnp.pad(fc_b[:, None], ((0, 8 - OUT), (0, 0)))
    bih = b_ih[:, None]
    bhh = b_hh[:, None]
    cb = cheb_b[:, None]

    def full(shape):
        return pl.BlockSpec(shape, lambda t: (0,) * len(shape))

    rec = functools.partial(_rec_kernel, B=B, NK=NK, T=T, HID=HID)
    out = pl.pallas_call(
        rec,
        grid=(T,),
        in_specs=[
            full((SP, SP)),
            pl.BlockSpec((1, B, CW, SP), lambda t: (t, 0, 0, 0)),
            pl.BlockSpec((1, B, CW, SP), lambda t: (t, 0, 0, 0)),
            full(W0.shape), full(W1.shape), full(Wx.shape), full(Wg.shape),
            full(Whh.shape), full(fc.shape),
            full(bih.shape), full(bhh.shape), full(cb.shape), full(fb.shape),
        ],
        out_specs=pl.BlockSpec((B, 8, SP), lambda t: (0, 0, 0)),
        out_shape=jax.ShapeDtypeStruct((B, 8, SP), f32),
        scratch_shapes=[
            pltpu.VMEM((B, HID, SP), f32),
            pltpu.VMEM((B, 8, SP), f32),
            pltpu.VMEM((B, SP), f32),
            pltpu.VMEM((B, SP), f32),
        ],
    )(AT, xall, pall, W0, W1, Wx, Wg, Whh, fc, bih, bhh, cb, fb)

    preds = out[:, :PRED, :S]
    return preds[..., None]


# speedup vs baseline: 46.1045x; 1.3578x over previous
import functools

import jax
import jax.numpy as jnp
from jax.experimental import pallas as pl
from jax.experimental.pallas import tpu as pltpu

_HI = jax.lax.Precision.HIGHEST


def _norm_kernel(dt_ref, at_ref):
    dt = dt_ref[...]
    deg = jnp.sum(dt, axis=1, keepdims=True)
    dis = jnp.where(deg > 0, 1.0 / jnp.sqrt(jnp.maximum(deg, 1e-12)), 0.0)
    at_ref[...] = -(dis * dt) * jnp.transpose(dis)


def _prop_kernel(at_ref, x_ref, p_ref):
    xb = x_ref[0]
    T, CW, SP = xb.shape
    res = jnp.dot(xb.reshape(T * CW, SP), at_ref[...],
                  preferred_element_type=jnp.float32, precision=_HI)
    p_ref[0] = res.reshape(T, CW, SP)


def _rec_kernel(at_ref, x_ref, p_ref,
                w01_ref, wxg_ref, whh_ref, fc_ref,
                bih_ref, bhh_ref, cb_ref, fb_ref,
                out_ref, h_ref, pr_ref, xct_ref, pct_ref, *, B, NK, T, HID):
    t = pl.program_id(0)
    f32 = jnp.float32
    SP = at_ref.shape[0]
    CW = x_ref.shape[2]
    pred = t >= NK

    @pl.when(t == 0)
    def _():
        h_ref[...] = jnp.zeros_like(h_ref)
        xct_ref[...] = jnp.zeros_like(xct_ref)
        pct_ref[...] = jnp.zeros_like(pct_ref)

    @pl.when(pred)
    def _():
        def fill(b, _):
            xc = (jnp.dot(fc_ref[...], h_ref[b], preferred_element_type=f32,
                          precision=_HI) + fb_ref[...])[0:1, :]
            xct_ref[pl.ds(b, 1), :] = xc
            return 0
        jax.lax.fori_loop(0, B, fill, 0)
        pct_ref[...] = jnp.dot(xct_ref[...], at_ref[...],
                               preferred_element_type=f32, precision=_HI)

    sub_x = jax.lax.broadcasted_iota(jnp.int32, (CW, SP), 0)
    sub8 = jax.lax.broadcasted_iota(jnp.int32, (8, SP), 0)
    i_out = t - (NK - 1)
    pm = jnp.where(pred, 1.0, 0.0).astype(f32)
    H = HID

    def body(b, _):
        h = h_ref[b]
        xb = x_ref[b, 0]
        pb = p_ref[b, 0]
        xcur = xct_ref[pl.ds(b, 1), :]
        pcb = pct_ref[pl.ds(b, 1), :]
        xf = jnp.where(jnp.logical_and(sub_x == 0, pred), xcur, xb)
        xp = jnp.concatenate([xf, pb], axis=0)
        xg = jax.nn.sigmoid(
            jnp.dot(w01_ref[...], xp, preferred_element_type=f32,
                    precision=_HI)
            + w01_ref[:, CW:CW + 1] * (pm * pcb) + cb_ref[...])
        gx = jnp.concatenate([xf, xg], axis=0)
        gi = (jnp.dot(wxg_ref[...], gx, preferred_element_type=f32,
                      precision=_HI)
              + bih_ref[...])
        gh = (jnp.dot(whh_ref[...], h, preferred_element_type=f32,
                      precision=_HI)
              + bhh_ref[...])
        r = jax.nn.sigmoid(gi[:H] + gh[:H])
        z = jax.nn.sigmoid(gi[H:2 * H] + gh[H:2 * H])
        n = jnp.tanh(gi[2 * H:] + r * gh[2 * H:])
        hn = (1.0 - z) * n + z * h
        h_ref[b] = hn
        xo = (jnp.dot(fc_ref[...], hn, preferred_element_type=f32,
                      precision=_HI) + fb_ref[...])[0:1, :]
        pr_ref[b] = jnp.where(sub8 == i_out, xo, pr_ref[b])
        return 0

    jax.lax.fori_loop(0, B, body, 0)

    @pl.when(t == T - 1)
    def _():
        out_ref[...] = pr_ref[...]


def kernel(x_hist, enc_misc, dec, edge_index, cheb_W, cheb_b,
           W_ih, W_hh, b_ih, b_hh, fc_W, fc_b):
    f32 = jnp.float32
    B, HIST, S, OUT = x_hist.shape
    FM = enc_misc.shape[-1]
    PRED = dec.shape[1]
    IN = OUT + FM
    GNN = cheb_W.shape[2]
    HID = W_hh.shape[1]
    NK = HIST
    NP = PRED - 1
    T = HIST - 1 + PRED
    SP = (S + 127) // 128 * 128
    CW = 16

    features = jnp.concatenate([enc_misc, dec], axis=1)

    xk = jnp.concatenate([x_hist, features[:, 1:NK + 1]], axis=-1)
    xk = jnp.pad(xk.transpose(0, 1, 3, 2),
                 ((0, 0), (0, 0), (0, CW - IN), (0, SP - S)))
    fp = jnp.pad(features[:, NK + 1:].transpose(0, 1, 3, 2),
                 ((0, 0), (0, 0), (1, CW - FM - 1), (0, SP - S)))
    xall = jnp.concatenate([xk, fp], axis=1)

    src, dst = edge_index[0], edge_index[1]
    w = (src != dst).astype(f32)
    DT = jnp.zeros((SP, SP), f32).at[src, dst].add(w)

    AT = pl.pallas_call(
        _norm_kernel,
        out_shape=jax.ShapeDtypeStruct((SP, SP), f32),
    )(DT)

    pall = pl.pallas_call(
        _prop_kernel,
        grid=(B,),
        in_specs=[
            pl.BlockSpec((SP, SP), lambda b: (0, 0)),
            pl.BlockSpec((1, T, CW, SP), lambda b: (b, 0, 0, 0)),
        ],
        out_specs=pl.BlockSpec((1, T, CW, SP), lambda b: (b, 0, 0, 0)),
        out_shape=jax.ShapeDtypeStruct((B, T, CW, SP), f32),
    )(AT, xall)

    W0 = jnp.pad(cheb_W[0].T, ((0, 0), (0, CW - IN)))
    W1 = jnp.pad(cheb_W[1].T, ((0, 0), (0, CW - IN)))
    w01 = jnp.concatenate([W0, W1], axis=1)
    Wx = jnp.pad(W_ih[:, :IN], ((0, 0), (0, CW - IN)))
    wxg = jnp.concatenate([Wx, W_ih[:, IN:]], axis=1)
    Whh = W_hh
    fc = jnp.pad(fc_W, ((0, 8 - OUT), (0, 0)))
    fb = jnp.pad(fc_b[:, None], ((0, 8 - OUT), (0, 0)))
    bih = b_ih[:, None]
    bhh = b_hh[:, None]
    cb = cheb_b[:, None]

    def full(shape):
        return pl.BlockSpec(shape, lambda t: (0,) * len(shape))

    rec = functools.partial(_rec_kernel, B=B, NK=NK, T=T, HID=HID)
    out = pl.pallas_call(
        rec,
        grid=(T,),
        in_specs=[
            full((SP, SP)),
            pl.BlockSpec((B, 1, CW, SP), lambda t: (0, t, 0, 0)),
            pl.BlockSpec((B, 1, CW, SP), lambda t: (0, t, 0, 0)),
            full(w01.shape), full(wxg.shape), full(Whh.shape), full(fc.shape),
            full(bih.shape), full(bhh.shape), full(cb.shape), full(fb.shape),
        ],
        out_specs=pl.BlockSpec((B, 8, SP), lambda t: (0, 0, 0)),
        out_shape=jax.ShapeDtypeStruct((B, 8, SP), f32),
        scratch_shapes=[
            pltpu.VMEM((B, HID, SP), f32),
            pltpu.VMEM((B, 8, SP), f32),
            pltpu.VMEM((B, SP), f32),
            pltpu.VMEM((B, SP), f32),
        ],
    )(AT, xall, pall, w01, wxg, Whh, fc, bih, bhh, cb, fb)

    preds = out[:, :PRED, :S]
    return preds[..., None]
